# baseline (speedup 1.0000x reference)
import jax
import jax.numpy as jnp
from jax import lax
from jax.experimental import pallas as pl
from jax.experimental.pallas import tpu as pltpu
from jax.experimental.pallas import tpu_sc as plsc

NPROP = 300
N_TOTAL = 5300
L = 16
NCHUNK = (N_TOTAL + L - 1) // L
NPAD = NCHUNK * L
OUT_W = 5
OUT_PAD = 1504
SCORE_THR = 0.85
NMS_THR = 0.1
NEG = -3.0e38
DONE_THR = -1.0e37

_f32 = jnp.float32
_i32 = jnp.int32


def _nms_body(ms_h, x1_h, y1_h, x2_h, y2_h, out_h,
              s_v, x1_v, y1_v, x2_v, y2_v,
              cidx, cs, cx1, cy1, cx2, cy2, car, outf):
    cid = lax.axis_index("c")
    sid = lax.axis_index("s")

    @pl.when(sid == 0)
    def _run():
        lanes = lax.iota(_i32, L)
        zi = jnp.broadcast_to(_i32(0), (L,))
        negv = jnp.broadcast_to(_f32(NEG), (L,))
        zf = jnp.broadcast_to(_f32(0.0), (L,))
        bigi = _i32(2147483647)

        pltpu.sync_copy(ms_h, s_v)
        pltpu.sync_copy(x1_h, x1_v)
        pltpu.sync_copy(y1_h, y1_v)
        pltpu.sync_copy(x2_h, x2_v)
        pltpu.sync_copy(y2_h, y2_v)

        @plsc.parallel_loop(0, NCHUNK, unroll=8)
        def _fill(i):
            cs[pl.ds(i * L, L)] = negv
            cidx[pl.ds(i * L, L)] = zi

        @plsc.parallel_loop(0, OUT_PAD // L, unroll=8)
        def _zout(i):
            outf[pl.ds(i * L, L)] = zf

        onef = jnp.broadcast_to(_f32(1.0), (L,))

        @plsc.parallel_loop(0, NCHUNK, unroll=4,
                            carry=(_f32(0.0), negv, zi))
        def _compact(i, carry):
            off, vmax, vidx = carry
            s = s_v[pl.ds(i * L, L)]
            gidx = lanes + i * L
            m = s > _f32(SCORE_THR)
            mf = jnp.where(m, onef, zf)
            cum = plsc.cumsum(mf)
            dest = (cum + (off - _f32(1.0))).astype(_i32)
            plsc.store_scatter(cidx, [dest], gidx, mask=m)
            plsc.store_scatter(cs, [dest], s, mask=m)
            better = s > vmax
            vmax = jnp.where(better, s, vmax)
            vidx = jnp.where(better, gidx, vidx)
            return off + jnp.max(cum), vmax, vidx

        offf, vmax, vidx = _compact
        nv = offf.astype(_i32)

        @pl.when(nv == 0)
        def _fallback():
            mx = jnp.max(vmax)
            sel = jnp.min(jnp.where(vmax == mx, vidx, bigi))
            lane0 = lanes == 0
            plsc.store_scatter(cidx, [zi], jnp.broadcast_to(sel, (L,)),
                               mask=lane0)
            plsc.store_scatter(cs, [zi], jnp.broadcast_to(mx, (L,)),
                               mask=lane0)

        nv1 = jnp.maximum(nv, 1)
        nch = (nv1 + (L - 1)) // L

        csf = jnp.where(cid == 0, _f32(1.25), _f32(1.0))
        bsf = jnp.where(cid == 0, _f32(1.0), _f32(1.0 / 1.25))

        @plsc.parallel_loop(0, nch, unroll=4, carry=(negv, zi))
        def _gather(i, c):
            bm, bi = c
            idxv = cidx[pl.ds(i * L, L)]
            sf = jnp.where(idxv < NPROP, csf, bsf)
            x1 = plsc.load_gather(x1_v, [idxv]) * sf
            y1 = plsc.load_gather(y1_v, [idxv]) * sf
            x2 = plsc.load_gather(x2_v, [idxv]) * sf
            y2 = plsc.load_gather(y2_v, [idxv]) * sf
            ar = (jnp.maximum(x2 - x1, _f32(0.0))
                  * jnp.maximum(y2 - y1, _f32(0.0)))
            cx1[pl.ds(i * L, L)] = x1
            cy1[pl.ds(i * L, L)] = y1
            cx2[pl.ds(i * L, L)] = x2
            cy2[pl.ds(i * L, L)] = y2
            car[pl.ds(i * L, L)] = ar
            s = cs[pl.ds(i * L, L)]
            ci = lanes + i * L
            better = s > bm
            return jnp.where(better, s, bm), jnp.where(better, ci, bi)

        bm0, bi0 = _gather
        mx0 = jnp.max(bm0)
        sel0 = jnp.min(jnp.where(bm0 == mx0, bi0, bigi))

        negone = jnp.broadcast_to(_i32(-1), (L,))

        def _cond(st):
            k, mx, sel, nc = st
            return (k < NPROP) & (mx > _f32(DONE_THR))

        def _iter(st):
            k, mx, sel, nc = st
            iv = jnp.broadcast_to(sel, (L,))
            x1s = plsc.load_gather(cx1, [iv])
            y1s = plsc.load_gather(cy1, [iv])
            x2s = plsc.load_gather(cx2, [iv])
            y2s = plsc.load_gather(cy2, [iv])
            ars = plsc.load_gather(car, [iv])
            mxv = jnp.broadcast_to(mx, (L,))
            row = jnp.where(lanes == 0, x1s,
                            jnp.where(lanes == 1, y1s,
                                      jnp.where(lanes == 2, x2s,
                                                jnp.where(lanes == 3,
                                                          y2s, mxv))))
            plsc.store_scatter(outf, [lanes + k * OUT_W], row,
                               mask=lanes < OUT_W)

            @plsc.parallel_loop(0, nc, unroll=8,
                                carry=(negv, zi, negone))
            def _sweep(i, c):
                bm, bi, lastc = c
                x1 = cx1[pl.ds(i * L, L)]
                y1 = cy1[pl.ds(i * L, L)]
                x2 = cx2[pl.ds(i * L, L)]
                y2 = cy2[pl.ds(i * L, L)]
                arc = car[pl.ds(i * L, L)]
                xx1 = jnp.maximum(x1s, x1)
                yy1 = jnp.maximum(y1s, y1)
                xx2 = jnp.minimum(x2s, x2)
                yy2 = jnp.minimum(y2s, y2)
                inter = (jnp.maximum(xx2 - xx1, _f32(0.0))
                         * jnp.maximum(yy2 - yy1, _f32(0.0)))
                denom = jnp.maximum(ars + arc - inter, _f32(1e-12))
                iou = inter / denom
                s = cs[pl.ds(i * L, L)]
                s_new = jnp.where(iou > _f32(NMS_THR), negv, s)
                cs[pl.ds(i * L, L)] = s_new
                ci = lanes + i * L
                better = s_new > bm
                bm = jnp.where(better, s_new, bm)
                bi = jnp.where(better, ci, bi)
                alive = s_new > _f32(DONE_THR)
                lastc = jnp.where(alive, jnp.broadcast_to(i, (L,)), lastc)
                return bm, bi, lastc

            bm, bi, lastc = _sweep
            mx2 = jnp.max(bm)
            sel2 = jnp.min(jnp.where(bm == mx2, bi, bigi))
            nc2 = jnp.max(lastc) + 1
            return k + 1, mx2, sel2, nc2

        lax.while_loop(_cond, _iter, (_i32(0), mx0, sel0, nch))

        pltpu.sync_copy(outf, out_h.at[cid])


def kernel(cached, bboxes, scores):
    cached = jnp.asarray(cached, _f32)
    bboxes = jnp.asarray(bboxes, _f32)
    scores = jnp.asarray(scores, _f32)

    pad = NPAD - N_TOTAL
    ms = jnp.concatenate([cached[:, 4], scores, jnp.full((pad,), NEG, _f32)])

    def col(j):
        return jnp.concatenate(
            [cached[:, j], bboxes[:, j], jnp.zeros((pad,), _f32)])

    mesh = plsc.VectorSubcoreMesh(core_axis_name="c", subcore_axis_name="s",
                                  num_cores=2, num_subcores=16)
    vec = lambda: pltpu.VMEM((NPAD,), _f32)
    out = pl.kernel(
        _nms_body,
        out_type=jax.ShapeDtypeStruct((2, OUT_PAD), _f32),
        mesh=mesh,
        compiler_params=pltpu.CompilerParams(needs_layout_passes=False),
        scratch_types=[
            vec(), vec(), vec(), vec(), vec(),
            pltpu.VMEM((NPAD,), _i32),
            vec(), vec(), vec(), vec(), vec(), vec(),
            pltpu.VMEM((OUT_PAD,), _f32),
        ],
    )(ms, col(0), col(1), col(2), col(3))
    return out[:, :NPROP * OUT_W].reshape(2, NPROP, OUT_W)


# speedup vs baseline: 1.0326x; 1.0326x over previous
import jax
import jax.numpy as jnp
from jax import lax
from jax.experimental import pallas as pl
from jax.experimental.pallas import tpu as pltpu
from jax.experimental.pallas import tpu_sc as plsc

NPROP = 300
N_TOTAL = 5300
L = 16
NCHUNK = (N_TOTAL + L - 1) // L
NPAD = NCHUNK * L
OUT_W = 5
OUT_PAD = 1504
SCORE_THR = 0.85
NMS_THR = 0.1
NEG = -3.0e38
DONE_THR = -1.0e37

_f32 = jnp.float32
_i32 = jnp.int32


def _nms_body(ms_h, x1_h, y1_h, x2_h, y2_h, out_h,
              s_v, x1_v, y1_v, x2_v, y2_v,
              cidx, cs, cx1, cy1, cx2, cy2, car, outf):
    cid = lax.axis_index("c")
    sid = lax.axis_index("s")

    @pl.when(sid == 0)
    def _run():
        lanes = lax.iota(_i32, L)
        zi = jnp.broadcast_to(_i32(0), (L,))
        negv = jnp.broadcast_to(_f32(NEG), (L,))
        zf = jnp.broadcast_to(_f32(0.0), (L,))
        bigi = _i32(2147483647)

        pltpu.sync_copy(ms_h, s_v)
        pltpu.sync_copy(x1_h, x1_v)
        pltpu.sync_copy(y1_h, y1_v)
        pltpu.sync_copy(x2_h, x2_v)
        pltpu.sync_copy(y2_h, y2_v)

        @plsc.parallel_loop(0, NCHUNK, unroll=8)
        def _fill(i):
            cs[pl.ds(i * L, L)] = negv
            cidx[pl.ds(i * L, L)] = zi

        @plsc.parallel_loop(0, OUT_PAD // L, unroll=8)
        def _zout(i):
            outf[pl.ds(i * L, L)] = zf

        onef = jnp.broadcast_to(_f32(1.0), (L,))

        @plsc.parallel_loop(0, NCHUNK, unroll=4,
                            carry=(_f32(0.0), negv, zi))
        def _compact(i, carry):
            off, vmax, vidx = carry
            s = s_v[pl.ds(i * L, L)]
            gidx = lanes + i * L
            m = s > _f32(SCORE_THR)
            mf = jnp.where(m, onef, zf)
            cum = plsc.cumsum(mf)
            dest = (cum + (off - _f32(1.0))).astype(_i32)
            plsc.store_scatter(cidx, [dest], gidx, mask=m)
            plsc.store_scatter(cs, [dest], s, mask=m)
            better = s > vmax
            vmax = jnp.where(better, s, vmax)
            vidx = jnp.where(better, gidx, vidx)
            return off + jnp.max(cum), vmax, vidx

        offf, vmax, vidx = _compact
        nv = offf.astype(_i32)

        @pl.when(nv == 0)
        def _fallback():
            mx = jnp.max(vmax)
            sel = jnp.min(jnp.where(vmax == mx, vidx, bigi))
            lane0 = lanes == 0
            plsc.store_scatter(cidx, [zi], jnp.broadcast_to(sel, (L,)),
                               mask=lane0)
            plsc.store_scatter(cs, [zi], jnp.broadcast_to(mx, (L,)),
                               mask=lane0)

        nv1 = jnp.maximum(nv, 1)
        nch = (nv1 + (L - 1)) // L

        csf = jnp.where(cid == 0, _f32(1.25), _f32(1.0))
        bsf = jnp.where(cid == 0, _f32(1.0), _f32(1.0 / 1.25))

        def _top2_update(s, ci, m1, i1, m2, i2):
            upd1 = s > m1
            upd2 = jnp.logical_not(upd1) & (s > m2)
            m2n = jnp.where(upd1, m1, jnp.where(upd2, s, m2))
            i2n = jnp.where(upd1, i1, jnp.where(upd2, ci, i2))
            return (jnp.where(upd1, s, m1), jnp.where(upd1, ci, i1),
                    m2n, i2n)

        def _extract_top2(m1, i1, m2, i2):
            g1 = jnp.max(m1)
            j1 = jnp.min(jnp.where(m1 == g1, i1, bigi))
            c1 = (m1 == g1) & (i1 == j1)
            cand = jnp.where(c1, m2, m1)
            candi = jnp.where(c1, i2, i1)
            g2 = jnp.max(cand)
            j2 = jnp.min(jnp.where(cand == g2, candi, bigi))
            return g1, j1, g2, j2

        @plsc.parallel_loop(0, nch, unroll=4,
                            carry=(negv, lanes, negv, lanes))
        def _gather(i, c2):
            idxv = cidx[pl.ds(i * L, L)]
            sf = jnp.where(idxv < NPROP, csf, bsf)
            x1 = plsc.load_gather(x1_v, [idxv]) * sf
            y1 = plsc.load_gather(y1_v, [idxv]) * sf
            x2 = plsc.load_gather(x2_v, [idxv]) * sf
            y2 = plsc.load_gather(y2_v, [idxv]) * sf
            ar = (jnp.maximum(x2 - x1, _f32(0.0))
                  * jnp.maximum(y2 - y1, _f32(0.0)))
            cx1[pl.ds(i * L, L)] = x1
            cy1[pl.ds(i * L, L)] = y1
            cx2[pl.ds(i * L, L)] = x2
            cy2[pl.ds(i * L, L)] = y2
            car[pl.ds(i * L, L)] = ar
            s = cs[pl.ds(i * L, L)]
            ci = lanes + i * L
            bm, bi, bm2, bi2 = c2
            return _top2_update(s, ci, bm, bi, bm2, bi2)

        M10, I10, M20, I20 = _extract_top2(*_gather)

        negone = jnp.broadcast_to(_i32(-1), (L,))

        def _cond(st):
            k, M1, I1, M2, I2, nc = st
            return (k < NPROP) & (M1 > _f32(DONE_THR))

        def _iter(st):
            k, M1, I1, M2, I2, nc = st
            sx1a = cx1[pl.ds(I1, L)][0]
            sy1a = cy1[pl.ds(I1, L)][0]
            sx2a = cx2[pl.ds(I1, L)][0]
            sy2a = cy2[pl.ds(I1, L)][0]
            sara = car[pl.ds(I1, L)][0]
            has2 = M2 > _f32(DONE_THR)
            I2g = jnp.where(has2, I2, _i32(0))
            sx1b = cx1[pl.ds(I2g, L)][0]
            sy1b = cy1[pl.ds(I2g, L)][0]
            sx2b = cx2[pl.ds(I2g, L)][0]
            sy2b = cy2[pl.ds(I2g, L)][0]
            sarb = car[pl.ds(I2g, L)][0]

            pint = (jnp.maximum(jnp.minimum(sx2a, sx2b)
                                - jnp.maximum(sx1a, sx1b), _f32(0.0))
                    * jnp.maximum(jnp.minimum(sy2a, sy2b)
                                  - jnp.maximum(sy1a, sy1b), _f32(0.0)))
            pden = jnp.maximum(sara + sarb - pint, _f32(1e-12))
            piouv = jnp.broadcast_to(pint, (L,)) / jnp.broadcast_to(pden,
                                                                    (L,))
            sv = jnp.max(jnp.where(piouv <= _f32(NMS_THR), onef, zf))
            ok2 = has2 & (sv > _f32(0.5)) & (k < NPROP - 1)
            x1a = jnp.broadcast_to(sx1a, (L,))
            y1a = jnp.broadcast_to(sy1a, (L,))
            x2a = jnp.broadcast_to(sx2a, (L,))
            y2a = jnp.broadcast_to(sy2a, (L,))
            ara = jnp.broadcast_to(sara, (L,))
            x1b = jnp.broadcast_to(sx1b, (L,))
            y1b = jnp.broadcast_to(sy1b, (L,))
            x2b = jnp.broadcast_to(sx2b, (L,))
            y2b = jnp.broadcast_to(sy2b, (L,))
            arb = jnp.broadcast_to(sarb, (L,))

            row1 = jnp.where(lanes == 0, x1a,
                             jnp.where(lanes == 1, y1a,
                                       jnp.where(lanes == 2, x2a,
                                                 jnp.where(lanes == 3, y2a,
                                                           jnp.broadcast_to(
                                                               M1, (L,))))))
            plsc.store_scatter(outf, [lanes + k * OUT_W], row1,
                               mask=lanes < OUT_W)

            @pl.when(ok2)
            def _emit2():
                row2 = jnp.where(lanes == 0, x1b,
                                 jnp.where(lanes == 1, y1b,
                                           jnp.where(lanes == 2, x2b,
                                                     jnp.where(
                                                         lanes == 3, y2b,
                                                         jnp.broadcast_to(
                                                             M2, (L,))))))
                plsc.store_scatter(outf, [lanes + (k + 1) * OUT_W], row2,
                                   mask=lanes < OUT_W)

            ok2v = jnp.broadcast_to(ok2, (L,))
            sx1b = jnp.where(ok2v, x1b, zf)
            sy1b = jnp.where(ok2v, y1b, zf)
            sx2b = jnp.where(ok2v, x2b, zf)
            sy2b = jnp.where(ok2v, y2b, zf)
            sarb = jnp.where(ok2v, arb, zf)

            @plsc.parallel_loop(0, nc, unroll=4,
                                carry=(negv, lanes, negv, lanes, negone))
            def _sweep(i, c):
                m1, i1, m2, i2, lastc = c
                x1 = cx1[pl.ds(i * L, L)]
                y1 = cy1[pl.ds(i * L, L)]
                x2 = cx2[pl.ds(i * L, L)]
                y2 = cy2[pl.ds(i * L, L)]
                arc = car[pl.ds(i * L, L)]
                inter1 = (jnp.maximum(jnp.minimum(x2a, x2) -
                                      jnp.maximum(x1a, x1), _f32(0.0))
                          * jnp.maximum(jnp.minimum(y2a, y2) -
                                        jnp.maximum(y1a, y1), _f32(0.0)))
                iou1 = inter1 / jnp.maximum(ara + arc - inter1, _f32(1e-12))
                inter2 = (jnp.maximum(jnp.minimum(sx2b, x2) -
                                      jnp.maximum(sx1b, x1), _f32(0.0))
                          * jnp.maximum(jnp.minimum(sy2b, y2) -
                                        jnp.maximum(sy1b, y1), _f32(0.0)))
                iou2 = inter2 / jnp.maximum(sarb + arc - inter2, _f32(1e-12))
                sup = (iou1 > _f32(NMS_THR)) | (iou2 > _f32(NMS_THR))
                s = cs[pl.ds(i * L, L)]
                s_new = jnp.where(sup, negv, s)
                cs[pl.ds(i * L, L)] = s_new
                ci = lanes + i * L
                m1, i1, m2, i2 = _top2_update(s_new, ci, m1, i1, m2, i2)
                alive = s_new > _f32(DONE_THR)
                lastc = jnp.where(alive, jnp.broadcast_to(i, (L,)), lastc)
                return m1, i1, m2, i2, lastc

            m1, i1, m2, i2, lastc = _sweep
            M1n, I1n, M2n, I2n = _extract_top2(m1, i1, m2, i2)
            nc2 = jnp.max(lastc) + 1
            kn = k + jnp.where(ok2, _i32(2), _i32(1))
            return kn, M1n, I1n, M2n, I2n, nc2

        lax.while_loop(_cond, _iter, (_i32(0), M10, I10, M20, I20, nch))

        pltpu.sync_copy(outf, out_h.at[cid])


def kernel(cached, bboxes, scores):
    cached = jnp.asarray(cached, _f32)
    bboxes = jnp.asarray(bboxes, _f32)
    scores = jnp.asarray(scores, _f32)

    pad = NPAD - N_TOTAL
    ms = jnp.concatenate([cached[:, 4], scores, jnp.full((pad,), NEG, _f32)])

    def col(j):
        return jnp.concatenate(
            [cached[:, j], bboxes[:, j], jnp.zeros((pad,), _f32)])

    mesh = plsc.VectorSubcoreMesh(core_axis_name="c", subcore_axis_name="s",
                                  num_cores=2, num_subcores=16)
    vec = lambda: pltpu.VMEM((NPAD,), _f32)
    cvec = lambda: pltpu.VMEM((NPAD + L,), _f32)
    out = pl.kernel(
        _nms_body,
        out_type=jax.ShapeDtypeStruct((2, OUT_PAD), _f32),
        mesh=mesh,
        compiler_params=pltpu.CompilerParams(needs_layout_passes=False),
        scratch_types=[
            vec(), vec(), vec(), vec(), vec(),
            pltpu.VMEM((NPAD,), _i32),
            vec(),
            cvec(), cvec(), cvec(), cvec(), cvec(),
            pltpu.VMEM((OUT_PAD,), _f32),
        ],
    )(ms, col(0), col(1), col(2), col(3))
    return out[:, :NPROP * OUT_W].reshape(2, NPROP, OUT_W)


# speedup vs baseline: 1.4399x; 1.3945x over previous
import jax
import jax.numpy as jnp
from jax import lax
from jax.experimental import pallas as pl
from jax.experimental.pallas import tpu as pltpu
from jax.experimental.pallas import tpu_sc as plsc

NPROP = 300
N_TOTAL = 5300
L = 16
NCHUNK = (N_TOTAL + L - 1) // L
NPAD = NCHUNK * L
OUT_W = 5
OUT_PAD = 1504
SCORE_THR = 0.85
NMS_THR = 0.1
NEG = -3.0e38
DONE_THR = -1.0e37

_f32 = jnp.float32
_i32 = jnp.int32


def _nms_body(ms_h, x1_h, y1_h, x2_h, y2_h, out_h,
              s_v, x1_v, y1_v, x2_v, y2_v,
              cidx, cs, cx1, cy1, cx2, cy2, car, outf, comm):
    cid = lax.axis_index("c")
    sid = lax.axis_index("s")

    @pl.when(sid == 0)
    def _run():
        lanes = lax.iota(_i32, L)
        zi = jnp.broadcast_to(_i32(0), (L,))
        negv = jnp.broadcast_to(_f32(NEG), (L,))
        zf = jnp.broadcast_to(_f32(0.0), (L,))
        bigi = _i32(2147483647)

        pltpu.sync_copy(ms_h, s_v)
        pltpu.sync_copy(x1_h, x1_v)
        pltpu.sync_copy(y1_h, y1_v)
        pltpu.sync_copy(x2_h, x2_v)
        pltpu.sync_copy(y2_h, y2_v)

        @plsc.parallel_loop(0, NCHUNK, unroll=8)
        def _fill(i):
            cs[pl.ds(i * L, L)] = negv
            cidx[pl.ds(i * L, L)] = zi

        @plsc.parallel_loop(0, OUT_PAD // L, unroll=8)
        def _zout(i):
            outf[pl.ds(i * L, L)] = zf

        onef = jnp.broadcast_to(_f32(1.0), (L,))

        @plsc.parallel_loop(0, NCHUNK, unroll=4,
                            carry=(_f32(0.0), negv, zi))
        def _compact(i, carry):
            off, vmax, vidx = carry
            s = s_v[pl.ds(i * L, L)]
            gidx = lanes + i * L
            m = s > _f32(SCORE_THR)
            mf = jnp.where(m, onef, zf)
            cum = plsc.cumsum(mf)
            dest = (cum + (off - _f32(1.0))).astype(_i32)
            plsc.store_scatter(cidx, [dest], gidx, mask=m)
            plsc.store_scatter(cs, [dest], s, mask=m)
            better = s > vmax
            vmax = jnp.where(better, s, vmax)
            vidx = jnp.where(better, gidx, vidx)
            return off + jnp.max(cum), vmax, vidx

        offf, vmax, vidx = _compact
        nv = offf.astype(_i32)

        @pl.when(nv == 0)
        def _fallback():
            mx = jnp.max(vmax)
            sel = jnp.min(jnp.where(vmax == mx, vidx, bigi))
            lane0 = lanes == 0
            plsc.store_scatter(cidx, [zi], jnp.broadcast_to(sel, (L,)),
                               mask=lane0)
            plsc.store_scatter(cs, [zi], jnp.broadcast_to(mx, (L,)),
                               mask=lane0)

        nv1 = jnp.maximum(nv, 1)
        nch = (nv1 + (L - 1)) // L

        csf = jnp.where(cid == 0, _f32(1.25), _f32(1.0))
        bsf = jnp.where(cid == 0, _f32(1.0), _f32(1.0 / 1.25))

        def _top2_update(s, ci, m1, i1, m2, i2):
            upd1 = s > m1
            upd2 = jnp.logical_not(upd1) & (s > m2)
            m2n = jnp.where(upd1, m1, jnp.where(upd2, s, m2))
            i2n = jnp.where(upd1, i1, jnp.where(upd2, ci, i2))
            return (jnp.where(upd1, s, m1), jnp.where(upd1, ci, i1),
                    m2n, i2n)

        def _extract_top2(m1, i1, m2, i2):
            g1 = jnp.max(m1)
            j1 = jnp.min(jnp.where(m1 == g1, i1, bigi))
            c1 = (m1 == g1) & (i1 == j1)
            cand = jnp.where(c1, m2, m1)
            candi = jnp.where(c1, i2, i1)
            g2 = jnp.max(cand)
            j2 = jnp.min(jnp.where(cand == g2, candi, bigi))
            return g1, j1, g2, j2

        @plsc.parallel_loop(0, nch, unroll=4,
                            carry=(negv, lanes, negv, lanes))
        def _gather(i, c2):
            idxv = cidx[pl.ds(i * L, L)]
            sf = jnp.where(idxv < NPROP, csf, bsf)
            x1 = plsc.load_gather(x1_v, [idxv]) * sf
            y1 = plsc.load_gather(y1_v, [idxv]) * sf
            x2 = plsc.load_gather(x2_v, [idxv]) * sf
            y2 = plsc.load_gather(y2_v, [idxv]) * sf
            ar = (jnp.maximum(x2 - x1, _f32(0.0))
                  * jnp.maximum(y2 - y1, _f32(0.0)))
            cx1[pl.ds(i * L, L)] = x1
            cy1[pl.ds(i * L, L)] = y1
            cx2[pl.ds(i * L, L)] = x2
            cy2[pl.ds(i * L, L)] = y2
            car[pl.ds(i * L, L)] = ar
            s = cs[pl.ds(i * L, L)]
            ci = lanes + i * L
            bm, bi, bm2, bi2 = c2
            return _top2_update(s, ci, bm, bi, bm2, bi2)

        M10, I10, M20, I20 = _extract_top2(*_gather)

        negone = jnp.broadcast_to(_i32(-1), (L,))

        def _cond(st):
            return (st[0] < NPROP) & (st[1] > _f32(DONE_THR))

        def _iter(st):
            k, M1, I1, M2, I2, nc, it = st
            sx1a = cx1[pl.ds(I1, L)][0]
            sy1a = cy1[pl.ds(I1, L)][0]
            sx2a = cx2[pl.ds(I1, L)][0]
            sy2a = cy2[pl.ds(I1, L)][0]
            sara = car[pl.ds(I1, L)][0]
            has2 = M2 > _f32(DONE_THR)
            I2g = jnp.where(has2, I2, _i32(0))
            sx1b = cx1[pl.ds(I2g, L)][0]
            sy1b = cy1[pl.ds(I2g, L)][0]
            sx2b = cx2[pl.ds(I2g, L)][0]
            sy2b = cy2[pl.ds(I2g, L)][0]
            sarb = car[pl.ds(I2g, L)][0]

            pint = (jnp.maximum(jnp.minimum(sx2a, sx2b)
                                - jnp.maximum(sx1a, sx1b), _f32(0.0))
                    * jnp.maximum(jnp.minimum(sy2a, sy2b)
                                  - jnp.maximum(sy1a, sy1b), _f32(0.0)))
            pden = jnp.maximum(sara + sarb - pint, _f32(1e-12))
            piouv = jnp.broadcast_to(pint, (L,)) / jnp.broadcast_to(pden,
                                                                    (L,))
            sv = jnp.max(jnp.where(piouv <= _f32(NMS_THR), onef, zf))
            ok2 = has2 & (sv > _f32(0.5)) & (k < NPROP - 1)
            x1a = jnp.broadcast_to(sx1a, (L,))
            y1a = jnp.broadcast_to(sy1a, (L,))
            x2a = jnp.broadcast_to(sx2a, (L,))
            y2a = jnp.broadcast_to(sy2a, (L,))
            ara = jnp.broadcast_to(sara, (L,))
            x1b = jnp.broadcast_to(sx1b, (L,))
            y1b = jnp.broadcast_to(sy1b, (L,))
            x2b = jnp.broadcast_to(sx2b, (L,))
            y2b = jnp.broadcast_to(sy2b, (L,))
            arb = jnp.broadcast_to(sarb, (L,))

            row1 = jnp.where(lanes == 0, x1a,
                             jnp.where(lanes == 1, y1a,
                                       jnp.where(lanes == 2, x2a,
                                                 jnp.where(lanes == 3, y2a,
                                                           jnp.broadcast_to(
                                                               M1, (L,))))))
            plsc.store_scatter(outf, [lanes + k * OUT_W], row1,
                               mask=lanes < OUT_W)

            @pl.when(ok2)
            def _emit2():
                row2 = jnp.where(lanes == 0, x1b,
                                 jnp.where(lanes == 1, y1b,
                                           jnp.where(lanes == 2, x2b,
                                                     jnp.where(
                                                         lanes == 3, y2b,
                                                         jnp.broadcast_to(
                                                             M2, (L,))))))
                plsc.store_scatter(outf, [lanes + (k + 1) * OUT_W], row2,
                                   mask=lanes < OUT_W)

            ok2v = jnp.broadcast_to(ok2, (L,))
            sx1b = jnp.where(ok2v, x1b, zf)
            sy1b = jnp.where(ok2v, y1b, zf)
            sx2b = jnp.where(ok2v, x2b, zf)
            sy2b = jnp.where(ok2v, y2b, zf)
            sarb = jnp.where(ok2v, arb, zf)

            @plsc.parallel_loop(0, nc, unroll=4,
                                carry=(negv, lanes, negv, lanes, negone))
            def _sweep(i, c):
                m1, i1, m2, i2, lastc = c
                x1 = cx1[pl.ds(i * L, L)]
                y1 = cy1[pl.ds(i * L, L)]
                x2 = cx2[pl.ds(i * L, L)]
                y2 = cy2[pl.ds(i * L, L)]
                arc = car[pl.ds(i * L, L)]
                inter1 = (jnp.maximum(jnp.minimum(x2a, x2) -
                                      jnp.maximum(x1a, x1), _f32(0.0))
                          * jnp.maximum(jnp.minimum(y2a, y2) -
                                        jnp.maximum(y1a, y1), _f32(0.0)))
                iou1 = inter1 / jnp.maximum(ara + arc - inter1, _f32(1e-12))
                inter2 = (jnp.maximum(jnp.minimum(sx2b, x2) -
                                      jnp.maximum(sx1b, x1), _f32(0.0))
                          * jnp.maximum(jnp.minimum(sy2b, y2) -
                                        jnp.maximum(sy1b, y1), _f32(0.0)))
                iou2 = inter2 / jnp.maximum(sarb + arc - inter2, _f32(1e-12))
                sup = (iou1 > _f32(NMS_THR)) | (iou2 > _f32(NMS_THR))
                s = cs[pl.ds(i * L, L)]
                s_new = jnp.where(sup, negv, s)
                cs[pl.ds(i * L, L)] = s_new
                ci = lanes + i * L
                m1, i1, m2, i2 = _top2_update(s_new, ci, m1, i1, m2, i2)
                alive = s_new > _f32(DONE_THR)
                lastc = jnp.where(alive, jnp.broadcast_to(i, (L,)), lastc)
                return m1, i1, m2, i2, lastc

            m1, i1, m2, i2, lastc = _sweep
            M1n, I1n, M2n, I2n = _extract_top2(m1, i1, m2, i2)
            nc2 = jnp.max(lastc) + 1
            kn = k + jnp.where(ok2, _i32(2), _i32(1))

            comm[pl.ds(0, L)] = jnp.where(lanes == 0,
                                          jnp.broadcast_to(_f32(-1.0), (L,)),
                                          zf)
            do_rc = ((it & _i32(7)) == _i32(7)) & (M1n > _f32(DONE_THR))

            @pl.when(do_rc)
            def _recompact():
                I1nv = jnp.broadcast_to(I1n, (L,))
                I2nv = jnp.broadcast_to(I2n, (L,))
                bigiv = jnp.broadcast_to(bigi, (L,))

                def _rc(i, c):
                    off, i1v, i2v = c
                    s = cs[pl.ds(i * L, L)]
                    m = s > _f32(DONE_THR)
                    mf = jnp.where(m, onef, zf)
                    cum = plsc.cumsum(mf)
                    dest = (cum + (off - _f32(1.0))).astype(_i32)
                    x1 = cx1[pl.ds(i * L, L)]
                    y1 = cy1[pl.ds(i * L, L)]
                    x2 = cx2[pl.ds(i * L, L)]
                    y2 = cy2[pl.ds(i * L, L)]
                    ar = car[pl.ds(i * L, L)]
                    plsc.store_scatter(cs, [dest], s, mask=m)
                    plsc.store_scatter(cx1, [dest], x1, mask=m)
                    plsc.store_scatter(cy1, [dest], y1, mask=m)
                    plsc.store_scatter(cx2, [dest], x2, mask=m)
                    plsc.store_scatter(cy2, [dest], y2, mask=m)
                    plsc.store_scatter(car, [dest], ar, mask=m)
                    ci = lanes + i * L
                    i1v = jnp.where((ci == I1nv) & m, dest, i1v)
                    i2v = jnp.where((ci == I2nv) & m, dest, i2v)
                    return off + jnp.max(cum), i1v, i2v

                off, i1v, i2v = lax.fori_loop(
                    0, nc2, _rc, (_f32(0.0), bigiv, bigiv))
                noff = off.astype(_i32)

                bch = noff >> 4

                @pl.when(bch < nc2)
                def _bwipe():
                    sb = cs[pl.ds(bch * L, L)]
                    cs[pl.ds(bch * L, L)] = jnp.where(
                        lanes >= (noff & _i32(15)), negv, sb)

                @plsc.parallel_loop(bch + 1, nc2, unroll=4)
                def _wipe(i):
                    cs[pl.ds(i * L, L)] = negv

                I1r = jnp.min(i1v)
                I2r = jnp.min(i2v)
                commv = jnp.where(
                    lanes == 0, jnp.broadcast_to(off, (L,)),
                    jnp.where(lanes == 1,
                              jnp.broadcast_to(I1r.astype(_f32), (L,)),
                              jnp.where(lanes == 2,
                                        jnp.broadcast_to(I2r.astype(_f32),
                                                         (L,)),
                                        zf)))
                comm[pl.ds(0, L)] = commv

            cvv = comm[pl.ds(0, L)]
            offr = cvv[0]
            got = offr >= _f32(0.0)
            nc3 = jnp.where(got, (offr.astype(_i32) + (L - 1)) >> 4, nc2)
            I1f = jnp.where(got, cvv[1].astype(_i32), I1n)
            I2f = jnp.where(got & (M2n > _f32(DONE_THR)),
                            cvv[2].astype(_i32), I2n)
            return kn, M1n, I1f, M2n, I2f, nc3, it + 1

        lax.while_loop(_cond, _iter,
                       (_i32(0), M10, I10, M20, I20, nch, _i32(0)))

        pltpu.sync_copy(outf, out_h.at[cid])


def kernel(cached, bboxes, scores):
    cached = jnp.asarray(cached, _f32)
    bboxes = jnp.asarray(bboxes, _f32)
    scores = jnp.asarray(scores, _f32)

    pad = NPAD - N_TOTAL
    ms = jnp.concatenate([cached[:, 4], scores, jnp.full((pad,), NEG, _f32)])

    def col(j):
        return jnp.concatenate(
            [cached[:, j], bboxes[:, j], jnp.zeros((pad,), _f32)])

    mesh = plsc.VectorSubcoreMesh(core_axis_name="c", subcore_axis_name="s",
                                  num_cores=2, num_subcores=16)
    vec = lambda: pltpu.VMEM((NPAD,), _f32)
    cvec = lambda: pltpu.VMEM((NPAD + L,), _f32)
    out = pl.kernel(
        _nms_body,
        out_type=jax.ShapeDtypeStruct((2, OUT_PAD), _f32),
        mesh=mesh,
        compiler_params=pltpu.CompilerParams(needs_layout_passes=False),
        scratch_types=[
            vec(), vec(), vec(), vec(), vec(),
            pltpu.VMEM((NPAD,), _i32),
            vec(),
            cvec(), cvec(), cvec(), cvec(), cvec(),
            pltpu.VMEM((OUT_PAD,), _f32),
            pltpu.VMEM((L,), _f32),
        ],
    )(ms, col(0), col(1), col(2), col(3))
    return out[:, :NPROP * OUT_W].reshape(2, NPROP, OUT_W)
